# baseline (speedup 1.0000x reference)
import functools

import jax
import jax.numpy as jnp
from jax import lax
from jax.experimental import pallas as pl
from jax.experimental.pallas import tpu as pltpu
from jax.experimental.pallas import tpu_sc as plsc

N = 10000
K = 32
D = 128
O = 128

NC = 2
NS = 16
NW = NC * NS
L = 16
N_PAD = 10240
NPT = N_PAD // NW
CH = 2
RG = CH * K
NCHUNK = NPT // CH
NB = 4
KU = 16

RB = 1024
RB2 = 1000


def _dense_pre_body(x_ref, pos_ref, wcat_ref, wp_ref, be_ref,
                    base_ref, c_ref, xw1_ref):
    y = jnp.dot(x_ref[...], wcat_ref[...], preferred_element_type=jnp.float32)
    p2 = (pos_ref[:, 0:1] * wp_ref[0:1, :] + pos_ref[:, 1:2] * wp_ref[1:2, :])
    base_ref[...] = (y[:, :O] - p2 + be_ref[...]).astype(jnp.bfloat16)
    c_ref[...] = (y[:, O:2 * O] + p2).astype(jnp.bfloat16)
    xw1_ref[...] = y[:, 2 * O:]


def _node_body(xw1_ref, agg_ref, wn2_ref, bn_ref, lns_ref, lnb_ref, mask_ref,
               out_ref):
    h = (xw1_ref[...]
         + jnp.dot(agg_ref[...].astype(jnp.float32), wn2_ref[...],
                   preferred_element_type=jnp.float32)
         + bn_ref[...])
    mu = jnp.mean(h, axis=1, keepdims=True)
    hc = h - mu
    var = jnp.mean(hc * hc, axis=1, keepdims=True)
    hn = hc * lax.rsqrt(var + 1e-5) * lns_ref[...] + lnb_ref[...]
    out_ref[...] = jnp.maximum(hn, 0.0) * mask_ref[...]


def _sc_gather_body(base_hbm, c_hbm, idx_hbm, out_hbm,
                    c_sp, idx_v, base_v, out_v, rows0, rows1, rows2, rows3,
                    sem0, sem1, sem2, sem3):
    cid = lax.axis_index("c")
    sid = lax.axis_index("s")
    wid = sid * NC + cid
    node0 = wid * NPT
    chunk0 = wid * NCHUNK

    rps = N_PAD // NS
    pltpu.sync_copy(c_hbm.at[pl.ds(sid * rps, rps)],
                    c_sp.at[pl.ds(sid * rps, rps)])

    pltpu.sync_copy(idx_hbm.at[pl.ds(chunk0, NCHUNK)], idx_v)
    pltpu.sync_copy(base_hbm.at[pl.ds(node0, NPT)], base_v)

    rows = (rows0, rows1, rows2, rows3)
    sems = (sem0, sem1, sem2, sem3)

    plsc.subcore_barrier()

    for b in range(NB):
        pltpu.async_copy(c_sp.at[idx_v.at[b]], rows[b], sems[b])

    W = 2 * L
    zero_bf = jnp.zeros((W,), jnp.bfloat16)

    def ring_body(it, carry):
        for b in range(NB):
            g = it * NB + b
            pltpu.make_async_copy(c_sp.at[idx_v.at[g]], rows[b],
                                  sems[b]).wait()
            rb = rows[b]
            for i in range(CH):
                nrow = g * CH + i
                bjs = [base_v[nrow, pl.ds(W * q, W)] for q in range(4)]

                def kg_body(kg, accs):
                    r0 = i * K + kg * KU
                    new = list(accs)
                    for q in range(4):
                        ms = [jnp.maximum(rb[r0 + kk, pl.ds(W * q, W)]
                                          + bjs[q], zero_bf)
                              for kk in range(KU)]
                        while len(ms) > 1:
                            ms = [ms[z] + ms[z + 1]
                                  for z in range(0, len(ms), 2)]
                        new[q] = new[q] + ms[0]
                    return tuple(new)

                accs = lax.fori_loop(
                    0, K // KU, kg_body,
                    tuple(jnp.zeros((W,), jnp.bfloat16) for _ in range(4)))
                for q in range(4):
                    out_v[nrow, pl.ds(W * q, W)] = accs[q]
            @pl.when(g + NB < NCHUNK)
            def _():
                pltpu.async_copy(c_sp.at[idx_v.at[g + NB]], rows[b], sems[b])
        return carry

    lax.fori_loop(0, NCHUNK // NB, ring_body, jnp.int32(0))

    pltpu.sync_copy(out_v, out_hbm.at[pl.ds(node0, NPT)])


_sc_gather = functools.partial(
    pl.kernel,
    out_type=jax.ShapeDtypeStruct((N_PAD, O), jnp.bfloat16),
    mesh=plsc.VectorSubcoreMesh(core_axis_name="c", subcore_axis_name="s",
                                num_cores=NC, num_subcores=NS),
    compiler_params=pltpu.CompilerParams(use_tc_tiling_on_sc=False),
    scratch_types=[
        pltpu.VMEM_SHARED((N_PAD, O), jnp.bfloat16),
        pltpu.VMEM((NCHUNK, RG), jnp.int32),
        pltpu.VMEM((NPT, O), jnp.bfloat16),
        pltpu.VMEM((NPT, O), jnp.bfloat16),
        pltpu.VMEM((RG, O), jnp.bfloat16),
        pltpu.VMEM((RG, O), jnp.bfloat16),
        pltpu.VMEM((RG, O), jnp.bfloat16),
        pltpu.VMEM((RG, O), jnp.bfloat16),
        pltpu.SemaphoreType.DMA,
        pltpu.SemaphoreType.DMA,
        pltpu.SemaphoreType.DMA,
        pltpu.SemaphoreType.DMA,
    ],
)(_sc_gather_body)


def kernel(x, pos, edge_idx, mask, W_e, b_e, W_n, b_n, ln_scale, ln_bias):
    B = x.shape[0]
    x2 = x.reshape(B * N, D)
    pos2 = pos.reshape(B * N, 2)
    pad = N_PAD - N
    idx_pad = jnp.pad(edge_idx.reshape(B * N, K).astype(jnp.int32),
                      ((0, pad), (0, 0))).reshape(NW * NCHUNK, RG)

    wcat = jnp.concatenate([W_e[:D], W_e[D:2 * D], W_n[:D]], axis=1)
    wn2 = W_n[D:] * jnp.float32(1.0 / K)

    grid = N_PAD // RB
    base_c_xw1 = pl.pallas_call(
        _dense_pre_body,
        grid=(grid,),
        in_specs=[
            pl.BlockSpec((RB, D), lambda i: (i, 0)),
            pl.BlockSpec((RB, 2), lambda i: (i, 0)),
            pl.BlockSpec((D, 3 * O), lambda i: (0, 0)),
            pl.BlockSpec((2, O), lambda i: (0, 0)),
            pl.BlockSpec((1, O), lambda i: (0, 0)),
        ],
        out_specs=[
            pl.BlockSpec((RB, O), lambda i: (i, 0)),
            pl.BlockSpec((RB, O), lambda i: (i, 0)),
            pl.BlockSpec((RB, O), lambda i: (i, 0)),
        ],
        out_shape=[
            jax.ShapeDtypeStruct((N_PAD, O), jnp.bfloat16),
            jax.ShapeDtypeStruct((N_PAD, O), jnp.bfloat16),
            jax.ShapeDtypeStruct((N_PAD, O), jnp.float32),
        ],
    )(x2, pos2, wcat, W_e[2 * D:], b_e.reshape(1, O))
    base, c_i32, xw1 = base_c_xw1

    agg = _sc_gather(base, c_i32, idx_pad)

    out = pl.pallas_call(
        _node_body,
        grid=(N // RB2,),
        in_specs=[
            pl.BlockSpec((RB2, O), lambda i: (i, 0)),
            pl.BlockSpec((RB2, O), lambda i: (i, 0)),
            pl.BlockSpec((O, O), lambda i: (0, 0)),
            pl.BlockSpec((1, O), lambda i: (0, 0)),
            pl.BlockSpec((1, O), lambda i: (0, 0)),
            pl.BlockSpec((1, O), lambda i: (0, 0)),
            pl.BlockSpec((RB2, 1), lambda i: (i, 0)),
        ],
        out_specs=pl.BlockSpec((RB2, O), lambda i: (i, 0)),
        out_shape=jax.ShapeDtypeStruct((N, O), jnp.float32),
    )(xw1, agg, wn2, b_n.reshape(1, O), ln_scale.reshape(1, O),
      ln_bias.reshape(1, O), mask.reshape(B * N, 1))

    return out.reshape(B, N, O)
